# Initial kernel scaffold; baseline (speedup 1.0000x reference)
import jax
import jax.numpy as jnp
from jax.experimental import pallas as pl


def kernel(hidden_states, gate_w, exp_gate, exp_up, exp_down, sh_gate, sh_up, sh_down):
    raise NotImplementedError("write your pallas kernel here")


# speedup vs baseline: 2.2356x; 2.2356x over previous
import functools

import jax
import jax.numpy as jnp
from jax.experimental import pallas as pl
from jax.experimental.pallas import tpu as pltpu


def _gelu_exact(x):
    return 0.5 * x * (1.0 + jax.lax.erf(x * 0.7071067811865476))


def _expert_mlp_body(x_ref, wg_ref, wu_ref, wd_ref, w_ref, y_ref):
    j = pl.program_id(1)
    x = x_ref[0]
    g = jax.lax.dot_general(x, wg_ref[0], (((1,), (1,)), ((), ())),
                            preferred_element_type=jnp.float32)
    u = jax.lax.dot_general(x, wu_ref[0], (((1,), (1,)), ((), ())),
                            preferred_element_type=jnp.float32)
    h = _gelu_exact(g) * u
    y = jax.lax.dot_general(h, wd_ref[0], (((1,), (1,)), ((), ())),
                            preferred_element_type=jnp.float32)

    @pl.when(j == 0)
    def _init():
        y_ref[...] = jnp.zeros_like(y_ref)

    y_ref[0] += y

    @pl.when(j == pl.num_programs(1) - 1)
    def _scale():
        y_ref[0] = y_ref[0] * w_ref[0, 0][:, None]


def _expert_mlp(xg, exp_gate, exp_up, exp_down, topk_w, *, jb=512):
    E, C, d = xg.shape
    ff = exp_gate.shape[1]
    nj = ff // jb
    return pl.pallas_call(
        _expert_mlp_body,
        grid=(E, nj),
        in_specs=[
            pl.BlockSpec((1, C, d), lambda e, j: (e, 0, 0)),
            pl.BlockSpec((1, jb, d), lambda e, j: (e, j, 0)),
            pl.BlockSpec((1, jb, d), lambda e, j: (e, j, 0)),
            pl.BlockSpec((1, d, jb), lambda e, j: (e, 0, j)),
            pl.BlockSpec((1, 1, C), lambda e, j: (e, 0, 0)),
        ],
        out_specs=pl.BlockSpec((1, C, d), lambda e, j: (e, 0, 0)),
        out_shape=jax.ShapeDtypeStruct((E, C, d), jnp.float32),
        compiler_params=pltpu.CompilerParams(
            dimension_semantics=("parallel", "arbitrary")),
    )(xg, exp_gate, exp_up, exp_down, topk_w.reshape(E, 1, C))


def _shared_mlp_body(x_ref, g_ref, u_ref, d_ref, o_ref):
    x = x_ref[...]
    g = jax.lax.dot_general(x, g_ref[...], (((1,), (1,)), ((), ())),
                            preferred_element_type=jnp.float32)
    u = jax.lax.dot_general(x, u_ref[...], (((1,), (1,)), ((), ())),
                            preferred_element_type=jnp.float32)
    h = _gelu_exact(g) * u
    o_ref[...] = jax.lax.dot_general(h, d_ref[...], (((1,), (1,)), ((), ())),
                                     preferred_element_type=jnp.float32)


def _shared_mlp(x, sh_gate, sh_up, sh_down, *, tb=512):
    N, d = x.shape
    sh = sh_gate.shape[0]
    nt = N // tb
    return pl.pallas_call(
        _shared_mlp_body,
        grid=(nt,),
        in_specs=[
            pl.BlockSpec((tb, d), lambda t: (t, 0)),
            pl.BlockSpec((sh, d), lambda t: (0, 0)),
            pl.BlockSpec((sh, d), lambda t: (0, 0)),
            pl.BlockSpec((d, sh), lambda t: (0, 0)),
        ],
        out_specs=pl.BlockSpec((tb, d), lambda t: (t, 0)),
        out_shape=jax.ShapeDtypeStruct((N, d), jnp.float32),
        compiler_params=pltpu.CompilerParams(
            dimension_semantics=("parallel",)),
    )(x, sh_gate, sh_up, sh_down)


def kernel(hidden_states, gate_w, exp_gate, exp_up, exp_down,
           sh_gate, sh_up, sh_down):
    B, S, d = hidden_states.shape
    E = gate_w.shape[0]
    N = B * S
    C = int(N * 2.0 / E)
    x = hidden_states.reshape(N, d)

    logits = x @ gate_w.T
    scores = jax.nn.softmax(logits, axis=-1)
    topk_w, topk_idx = jax.lax.top_k(scores.T, C)

    flat_idx = topk_idx.reshape(-1)
    xg = jnp.take(x, flat_idx, axis=0).reshape(E, C, d)
    y = _expert_mlp(xg, exp_gate, exp_up, exp_down, topk_w)

    out = _shared_mlp(x, sh_gate, sh_up, sh_down)
    out = out.at[flat_idx].add(y.reshape(N * 2, d))
    return out.reshape(B, S, d)
